# baseline (speedup 1.0000x reference)
import functools

import jax
import jax.numpy as jnp
from jax import lax
from jax.experimental import pallas as pl
from jax.experimental.pallas import tpu as pltpu
from jax.experimental.pallas import tpu_sc as plsc

_VOCAB = 1000000
_EMBED = 64
_NCLS = 10
_B = 4096
_L = 200
_LP = 208
_HALF = _LP // 2
_NC = 2
_NS = 16
_NW = _NC * _NS
_SEQ_PER_W = _B // _NW
_LANES = 16


def _sc_body(table_hbm, idx_hbm, out_hbm, idx_v, rows_v, outbuf, sem):
    wid = lax.axis_index("s") * _NC + lax.axis_index("c")
    base = wid * _SEQ_PER_W
    pltpu.sync_copy(idx_hbm.at[pl.ds(base, _SEQ_PER_W)], idx_v)

    nvec = _EMBED // _LANES

    def seq_body(s, carry):
        accs = tuple(jnp.zeros((_LANES,), jnp.float32) for _ in range(nvec))
        for h in range(2):
            pltpu.async_copy(table_hbm.at[idx_v.at[s, h]], rows_v, sem).wait()

            def row_body(r, accs):
                return tuple(
                    a + rows_v[r, pl.ds(e * _LANES, _LANES)]
                    for e, a in enumerate(accs)
                )

            accs = lax.fori_loop(0, _HALF, row_body, accs)
        for e in range(nvec):
            outbuf[s, pl.ds(e * _LANES, _LANES)] = accs[e]
        return carry

    lax.fori_loop(0, _SEQ_PER_W, seq_body, 0)
    pltpu.sync_copy(outbuf, out_hbm.at[pl.ds(base, _SEQ_PER_W)])


_sc_gather_sum = functools.partial(
    pl.kernel,
    out_type=jax.ShapeDtypeStruct((_B, _EMBED), jnp.float32),
    mesh=plsc.VectorSubcoreMesh(
        core_axis_name="c", subcore_axis_name="s",
        num_cores=_NC, num_subcores=_NS),
    scratch_types=[
        pltpu.VMEM((_SEQ_PER_W, 2, _HALF), jnp.int32),
        pltpu.VMEM((_HALF, _EMBED), jnp.float32),
        pltpu.VMEM((_SEQ_PER_W, _EMBED), jnp.float32),
        pltpu.SemaphoreType.DMA,
    ],
    compiler_params=pltpu.CompilerParams(use_tc_tiling_on_sc=False),
)(_sc_body)


def _tc_body(summed_ref, seq_ref, wt_ref, b_ref, out_ref):
    cnt = jnp.sum((seq_ref[...] != 0).astype(jnp.float32), axis=1,
                  keepdims=True)
    cnt = jnp.maximum(cnt, 1.0)
    acc = jnp.dot(summed_ref[...], wt_ref[...],
                  preferred_element_type=jnp.float32)
    out_ref[...] = acc / cnt + b_ref[...]


def _tc_classifier(summed, seqs, wt, b2):
    blk = 1024
    grid = _B // blk
    return pl.pallas_call(
        _tc_body,
        grid=(grid,),
        in_specs=[
            pl.BlockSpec((blk, _EMBED), lambda i: (i, 0)),
            pl.BlockSpec((blk, _L), lambda i: (i, 0)),
            pl.BlockSpec((_EMBED, _NCLS), lambda i: (0, 0)),
            pl.BlockSpec((1, _NCLS), lambda i: (0, 0)),
        ],
        out_specs=pl.BlockSpec((blk, _NCLS), lambda i: (i, 0)),
        out_shape=jax.ShapeDtypeStruct((_B, _NCLS), jnp.float32),
    )(summed, seqs, wt, b2)


def kernel(sequences, emb_table, W, b):
    seqs = sequences.astype(jnp.int32)
    idx3 = jnp.pad(seqs, ((0, 0), (0, _LP - _L))).reshape(_B, 2, _HALF)
    summed = _sc_gather_sum(emb_table, idx3)
    return _tc_classifier(summed, seqs, W.T, b.reshape(1, _NCLS))
